# baseline (speedup 1.0000x reference)
import functools

import jax
import jax.numpy as jnp
from jax import lax
from jax.experimental import pallas as pl
from jax.experimental.pallas import tpu as pltpu

_NEG_INF = float("-inf")


def _fused_kernel(qi_ref, wq_ref, bq_ref, keys_ref, vals_ref, iw_ref,
                  wa_ref, ba_ref, wc_ref, bc_ref, out_ref,
                  qn_ref, rv_ref, ri_ref, rrv_ref, *, C, K, nchunks):
    c = pl.program_id(1)
    B = qi_ref.shape[0]
    D = wq_ref.shape[0]

    @pl.when(c == 0)
    def _init():
        q = jnp.dot(qi_ref[:], wq_ref[:].T,
                    preferred_element_type=jnp.float32) + bq_ref[:]
        nrm = jnp.sqrt(jnp.sum(q * q, axis=1, keepdims=True))
        qn_ref[:] = q / jnp.maximum(nrm, 1e-12)
        rv_ref[:] = jnp.full((B, K), _NEG_INF, jnp.float32)
        ri_ref[:] = jnp.zeros((B, K), jnp.int32)
        rrv_ref[:] = jnp.zeros((B, K * D), jnp.float32)

    kb = keys_ref[:]
    knrm = jnp.sqrt(jnp.sum(kb * kb, axis=1, keepdims=True))
    kn = kb / jnp.maximum(knrm, 1e-12)
    scores = jnp.dot(qn_ref[:], kn.T,
                     preferred_element_type=jnp.float32) * iw_ref[0]

    iota = lax.broadcasted_iota(jnp.int32, (B, C), 1)
    vb = vals_ref[:]
    base = c * C
    cols_v = [rv_ref[:, j:j + 1] for j in range(K)]
    cols_i = [ri_ref[:, j:j + 1] for j in range(K)]
    cols_rv = [rrv_ref[:, j * D:(j + 1) * D] for j in range(K)]
    s = scores
    for _ in range(K):
        m = jnp.max(s, axis=1, keepdims=True)
        eq = s == m
        loc = jnp.min(jnp.where(eq, iota, jnp.int32(C)),
                      axis=1, keepdims=True)
        onehot = iota == loc
        v_row = jnp.dot(jnp.where(onehot, 1.0, 0.0), vb,
                        preferred_element_type=jnp.float32)
        cols_v.append(m)
        cols_i.append(loc + base)
        cols_rv.append(v_row)
        s = jnp.where(onehot, _NEG_INF, s)

    big_i = jnp.int32(2 ** 30)
    vcur = list(cols_v)
    new_v, new_i, new_rv = [], [], []
    for _ in range(K):
        m = functools.reduce(jnp.maximum, vcur)
        pick = functools.reduce(
            jnp.minimum,
            [jnp.where(v == m, i, big_i) for v, i in zip(vcur, cols_i)])
        sels = [(v == m) & (i == pick) for v, i in zip(vcur, cols_i)]
        rv = functools.reduce(
            jnp.add,
            [jnp.where(sel, r, 0.0) for sel, r in zip(sels, cols_rv)])
        new_v.append(m)
        new_i.append(pick)
        new_rv.append(rv)
        vcur = [jnp.where(sel, _NEG_INF, v) for sel, v in zip(sels, vcur)]

    rv_ref[:] = jnp.concatenate(new_v, axis=1)
    ri_ref[:] = jnp.concatenate(new_i, axis=1)
    rrv_ref[:] = jnp.concatenate(new_rv, axis=1)

    @pl.when(c == nchunks - 1)
    def _finalize():
        wa = wa_ref[:]
        ba0 = ba_ref[0, 0]
        logits = [jnp.sum(r * wa, axis=1, keepdims=True) + ba0 for r in new_rv]
        mx = functools.reduce(jnp.maximum, logits)
        es = [jnp.exp(l - mx) for l in logits]
        tot = functools.reduce(jnp.add, es)
        mem = functools.reduce(
            jnp.add, [e * r for e, r in zip(es, new_rv)]) / tot
        out_ref[:] = jnp.dot(mem, wc_ref[:].T,
                             preferred_element_type=jnp.float32) + bc_ref[:]


def kernel(query_input, memory_keys, memory_values, importance_weights,
           Wq, bq, Wa, ba, Wc, bc, top_k):
    B, CS = query_input.shape
    M, D = memory_keys.shape
    K = 8
    C = next(c for c in range(min(M, 2048), 0, -8) if M % c == 0)
    nchunks = M // C
    BB = next(b for b in (256, 128, 64, 32, 16, 8, B) if B % b == 0)
    nb = B // BB

    iw2 = importance_weights.reshape(nchunks, 1, C)
    bq2 = bq.reshape(1, D)
    ba2 = ba.reshape(1, 1)
    bc2 = bc.reshape(1, D)

    return pl.pallas_call(
        functools.partial(_fused_kernel, C=C, K=K, nchunks=nchunks),
        grid=(nb, nchunks),
        in_specs=[
            pl.BlockSpec((BB, CS), lambda b, c: (b, 0)),
            pl.BlockSpec((D, CS), lambda b, c: (0, 0)),
            pl.BlockSpec((1, D), lambda b, c: (0, 0)),
            pl.BlockSpec((C, D), lambda b, c: (c, 0)),
            pl.BlockSpec((C, D), lambda b, c: (c, 0)),
            pl.BlockSpec((1, 1, C), lambda b, c: (c, 0, 0)),
            pl.BlockSpec((1, D), lambda b, c: (0, 0)),
            pl.BlockSpec((1, 1), lambda b, c: (0, 0)),
            pl.BlockSpec((D, D), lambda b, c: (0, 0)),
            pl.BlockSpec((1, D), lambda b, c: (0, 0)),
        ],
        out_specs=pl.BlockSpec((BB, D), lambda b, c: (b, 0)),
        out_shape=jax.ShapeDtypeStruct((B, D), jnp.float32),
        scratch_shapes=[
            pltpu.VMEM((BB, D), jnp.float32),
            pltpu.VMEM((BB, K), jnp.float32),
            pltpu.VMEM((BB, K), jnp.int32),
            pltpu.VMEM((BB, K * D), jnp.float32),
        ],
        compiler_params=pltpu.CompilerParams(
            dimension_semantics=("arbitrary", "arbitrary"),
        ),
    )(query_input, Wq, bq2, memory_keys, memory_values, iw2, Wa, ba2, Wc, bc2)


# speedup vs baseline: 1.4480x; 1.4480x over previous
import functools

import jax
import jax.numpy as jnp
from jax import lax
from jax.experimental import pallas as pl
from jax.experimental.pallas import tpu as pltpu
from jax.experimental.pallas import tpu_sc as plsc

_NEG_INF = float("-inf")


def _select_kernel(qi_ref, wq_ref, bq_ref, keys_ref, iw_ref, idx_out_ref,
                   qn_ref, rv_ref, ri_ref, *, C, K, nchunks):
    c = pl.program_id(1)
    B = qi_ref.shape[0]

    @pl.when(c == 0)
    def _init():
        q = jnp.dot(qi_ref[:], wq_ref[:].T,
                    preferred_element_type=jnp.float32) + bq_ref[:]
        nrm = jnp.sqrt(jnp.sum(q * q, axis=1, keepdims=True))
        qn_ref[:] = q / jnp.maximum(nrm, 1e-12)
        rv_ref[:] = jnp.full((B, K), _NEG_INF, jnp.float32)
        ri_ref[:] = jnp.zeros((B, K), jnp.int32)

    kb = keys_ref[:]
    knrm = jnp.sqrt(jnp.sum(kb * kb, axis=1, keepdims=True))
    kn = kb / jnp.maximum(knrm, 1e-12)
    scores = jnp.dot(qn_ref[:], kn.T,
                     preferred_element_type=jnp.float32) * iw_ref[0]

    iota = lax.broadcasted_iota(jnp.int32, (B, C), 1)
    base = c * C
    cols_v = [rv_ref[:, j:j + 1] for j in range(K)]
    cols_i = [ri_ref[:, j:j + 1] for j in range(K)]
    s = scores
    for _ in range(K):
        loc = jnp.argmax(s, axis=1, keepdims=True)
        m = jnp.max(s, axis=1, keepdims=True)
        onehot = iota == loc
        cols_v.append(m)
        cols_i.append(loc.astype(jnp.int32) + base)
        s = jnp.where(onehot, _NEG_INF, s)

    big_i = jnp.int32(2 ** 30)
    vcur = list(cols_v)
    new_v, new_i = [], []
    for _ in range(K):
        m = functools.reduce(jnp.maximum, vcur)
        pick = functools.reduce(
            jnp.minimum,
            [jnp.where(v == m, i, big_i) for v, i in zip(vcur, cols_i)])
        sels = [(v == m) & (i == pick) for v, i in zip(vcur, cols_i)]
        new_v.append(m)
        new_i.append(pick)
        vcur = [jnp.where(sel, _NEG_INF, v) for sel, v in zip(sels, vcur)]

    rv_ref[:] = jnp.concatenate(new_v, axis=1)
    ri_ref[:] = jnp.concatenate(new_i, axis=1)

    @pl.when(c == nchunks - 1)
    def _emit():
        idx_out_ref[:] = jnp.concatenate(new_i, axis=1)


def _sc_gather_body(table_hbm, idx_hbm, out_hbm, idx_v, rows_v, sem,
                    *, NC, b_per_w):
    wid = lax.axis_index("s") * NC + lax.axis_index("c")
    base = wid * b_per_w
    pltpu.sync_copy(idx_hbm.at[pl.ds(base, b_per_w)], idx_v)
    pltpu.async_copy(table_hbm.at[idx_v], rows_v, sem).wait()
    pltpu.sync_copy(rows_v, out_hbm.at[pl.ds(base, b_per_w)])


def _attn_kernel(rv_ref, wa_ref, ba_ref, wc_ref, bc_ref, out_ref, *, K, D):
    wa = wa_ref[:]
    ba0 = ba_ref[0, 0]
    rows = [rv_ref[:, j * D:(j + 1) * D] for j in range(K)]
    logits = [jnp.sum(r * wa, axis=1, keepdims=True) + ba0 for r in rows]
    mx = functools.reduce(jnp.maximum, logits)
    es = [jnp.exp(l - mx) for l in logits]
    tot = functools.reduce(jnp.add, es)
    mem = functools.reduce(
        jnp.add, [e * r for e, r in zip(es, rows)]) / tot
    out_ref[:] = jnp.dot(mem, wc_ref[:].T,
                         preferred_element_type=jnp.float32) + bc_ref[:]


def kernel(query_input, memory_keys, memory_values, importance_weights,
           Wq, bq, Wa, ba, Wc, bc, top_k):
    B, CS = query_input.shape
    M, D = memory_keys.shape
    K = 8
    C = next(c for c in range(min(M, 2048), 0, -8) if M % c == 0)
    nchunks = M // C
    BB = next(b for b in (256, 128, 64, 32, 16, 8, B) if B % b == 0)
    nb = B // BB

    iw2 = importance_weights.reshape(nchunks, 1, C)
    bq2 = bq.reshape(1, D)
    ba2 = ba.reshape(1, 1)
    bc2 = bc.reshape(1, D)

    top_idx = pl.pallas_call(
        functools.partial(_select_kernel, C=C, K=K, nchunks=nchunks),
        grid=(nb, nchunks),
        in_specs=[
            pl.BlockSpec((BB, CS), lambda b, c: (b, 0)),
            pl.BlockSpec((D, CS), lambda b, c: (0, 0)),
            pl.BlockSpec((1, D), lambda b, c: (0, 0)),
            pl.BlockSpec((C, D), lambda b, c: (c, 0)),
            pl.BlockSpec((1, 1, C), lambda b, c: (c, 0, 0)),
        ],
        out_specs=pl.BlockSpec((BB, K), lambda b, c: (b, 0)),
        out_shape=jax.ShapeDtypeStruct((B, K), jnp.int32),
        scratch_shapes=[
            pltpu.VMEM((BB, D), jnp.float32),
            pltpu.VMEM((BB, K), jnp.float32),
            pltpu.VMEM((BB, K), jnp.int32),
        ],
        compiler_params=pltpu.CompilerParams(
            dimension_semantics=("arbitrary", "arbitrary"),
        ),
    )(query_input, Wq, bq2, memory_keys, iw2)

    info = plsc.get_sparse_core_info()
    NC, NS = info.num_cores, info.num_subcores
    NW = NC * NS
    b_per_w = (B * K) // NW
    flat_idx = top_idx.reshape(B * K)
    mesh = plsc.VectorSubcoreMesh(core_axis_name="c", subcore_axis_name="s")
    gathered = pl.kernel(
        functools.partial(_sc_gather_body, NC=NC, b_per_w=b_per_w),
        mesh=mesh,
        out_type=jax.ShapeDtypeStruct((B * K, D), jnp.float32),
        scratch_types=[
            pltpu.VMEM((b_per_w,), jnp.int32),
            pltpu.VMEM((b_per_w, D), jnp.float32),
            pltpu.SemaphoreType.DMA,
        ],
        compiler_params=pltpu.CompilerParams(use_tc_tiling_on_sc=False),
    )(memory_values, flat_idx)

    rv = gathered.reshape(B, K * D)
    return pl.pallas_call(
        functools.partial(_attn_kernel, K=K, D=D),
        out_shape=jax.ShapeDtypeStruct((B, D), jnp.float32),
    )(rv, Wa, ba2, Wc, bc2)
